# Initial kernel scaffold; baseline (speedup 1.0000x reference)
import jax
import jax.numpy as jnp
from jax.experimental import pallas as pl


def kernel(h_self, h_neigh, indptr, indices, W_neigh, W_self, b_self):
    raise NotImplementedError("write your pallas kernel here")


# speedup vs baseline: 111.7475x; 111.7475x over previous
import dataclasses
import functools

import jax
import jax.numpy as jnp
from jax import lax
from jax.experimental import pallas as pl
from jax.experimental.pallas import tpu as pltpu
from jax.experimental.pallas import tpu_sc as plsc

NC = 2
NS = 16
L = 16

BE = 128
IPLEN = 16384
ZR = 64


def _sc_compiler_params():
    cp = pltpu.CompilerParams()
    if "needs_layout_passes" in pltpu.CompilerParams.__dataclass_fields__:
        cp = dataclasses.replace(cp, needs_layout_passes=False)
    return cp


def _sc_aggregate(h_neigh, indices, ip_pad, n_nodes, n_pad):
    E = indices.shape[0]
    D = h_neigh.shape[1]
    nb_total = E // BE
    per_core = nb_total // NC
    nb_base = per_core // NS
    nb_rem = per_core % NS
    rows_per_sub = n_pad // NS
    n_zero_copies = rows_per_sub // ZR

    mesh = plsc.VectorSubcoreMesh(core_axis_name="c", subcore_axis_name="s")

    @functools.partial(
        pl.kernel,
        out_type=jax.ShapeDtypeStruct((NC, n_pad, D), jnp.float32),
        mesh=mesh,
        scratch_types=[
            pltpu.VMEM((IPLEN,), jnp.int32),
            pltpu.VMEM((BE,), jnp.int32),
            pltpu.VMEM((BE, D), jnp.float32),
            pltpu.VMEM((BE,), jnp.int32),
            pltpu.VMEM((ZR, D), jnp.float32),
            pltpu.VMEM_SHARED((n_pad, D), jnp.float32),
        ],
        compiler_params=_sc_compiler_params(),
    )
    def agg(h_hbm, idx_hbm, ip_hbm, out_hbm, ip_v, idx_v, rows_v, dst_v,
            zero_v, acc_sh):
        cid = lax.axis_index("c")
        sid = lax.axis_index("s")

        pltpu.sync_copy(ip_hbm, ip_v)

        @pl.loop(0, ZR)
        def _(r):
            for j in range(D // L):
                zero_v[r, pl.ds(j * L, L)] = jnp.zeros((L,), jnp.float32)

        base_row = sid * rows_per_sub
        for k in range(n_zero_copies):
            pltpu.sync_copy(zero_v, acc_sh.at[pl.ds(base_row + k * ZR, ZR)])
        plsc.subcore_barrier()

        nblocks = jnp.where(sid < nb_rem, nb_base + 1, nb_base)

        @pl.loop(0, nblocks)
        def _(i):
            b = cid * per_core + sid + i * NS
            e0 = b * BE
            pltpu.sync_copy(idx_hbm.at[pl.ds(e0, BE)], idx_v)
            pltpu.sync_copy(h_hbm.at[idx_v], rows_v)

            @pl.loop(0, BE // L)
            def _(g):
                e_vec = lax.iota(jnp.int32, L) + (e0 + g * L)
                base = jnp.zeros((L,), jnp.int32)
                step = IPLEN // 2
                while step >= 1:
                    mid = base + step
                    a = plsc.load_gather(ip_v, [mid - 1])
                    base = jnp.where(a <= e_vec, mid, base)
                    step //= 2
                dst = base - 1
                ok = (dst >= 0) & (dst < n_nodes)
                dst_v[pl.ds(g * L, L)] = jnp.where(ok, dst, n_nodes)

            pltpu.sync_copy(rows_v, acc_sh.at[dst_v], add=True)

        plsc.subcore_barrier()
        pltpu.sync_copy(acc_sh.at[pl.ds(base_row, rows_per_sub)],
                        out_hbm.at[cid, pl.ds(base_row, rows_per_sub)])

    return agg(h_neigh, indices, ip_pad)


def _tc_finish_body(s0_ref, s1_ref, hs_ref, lo_ref, hi_ref, wn_ref, ws_ref,
                    b_ref, out_ref):
    deg = jnp.maximum((hi_ref[...] - lo_ref[...]).astype(jnp.float32), 1.0)
    h_agg = (s0_ref[...] + s1_ref[...]) / deg
    dn = (((1,), (1,)), ((), ()))
    out_ref[...] = (
        lax.dot_general(h_agg, wn_ref[...], dn,
                        preferred_element_type=jnp.float32)
        + lax.dot_general(hs_ref[...], ws_ref[...], dn,
                          preferred_element_type=jnp.float32)
        + b_ref[...]
    )


def _tc_finish(s0, s1, h_self, ip_lo, ip_hi, W_neigh, W_self, b2):
    N, D = h_self.shape
    OUT = W_neigh.shape[0]
    BN = 2000
    grid = (N // BN,)
    row_spec = pl.BlockSpec((BN, D), lambda i: (i, 0))
    col1_spec = pl.BlockSpec((BN, 1), lambda i: (i, 0))
    full = lambda shape: pl.BlockSpec(shape, lambda i: (0, 0))
    return pl.pallas_call(
        _tc_finish_body,
        grid=grid,
        in_specs=[row_spec, row_spec, row_spec, col1_spec, col1_spec,
                  full((OUT, D)), full((OUT, D)), full((1, OUT))],
        out_specs=pl.BlockSpec((BN, OUT), lambda i: (i, 0)),
        out_shape=jax.ShapeDtypeStruct((N, OUT), jnp.float32),
    )(s0, s1, h_self, ip_lo, ip_hi, W_neigh, W_self, b2)


def kernel(h_self, h_neigh, indptr, indices, W_neigh, W_self, b_self):
    N, D = h_self.shape
    n_pad = ((N + 1) + (NS * ZR) - 1) // (NS * ZR) * (NS * ZR)
    ip_pad = jnp.concatenate([
        indptr.astype(jnp.int32),
        jnp.full((IPLEN - (N + 1),), jnp.iinfo(jnp.int32).max, jnp.int32),
    ])
    acc = _sc_aggregate(h_neigh, indices, ip_pad, N, n_pad)
    ip_lo = indptr[:-1].reshape(N, 1)
    ip_hi = indptr[1:].reshape(N, 1)
    b2 = b_self.reshape(1, -1)
    return _tc_finish(acc[0, :N], acc[1, :N], h_self, ip_lo, ip_hi,
                      W_neigh, W_self, b2)
